# baseline (speedup 1.0000x reference)
import jax
import jax.numpy as jnp
from jax.experimental import pallas as pl
from jax.experimental.pallas import tpu as pltpu
from jax.experimental.pallas import tpu_sc as plsc

_VOCAB = 100000
_EMBED = 64
_BATCH = 1024

_V_TILE = 2048

_SC_CORES = 2
_SC_SUBCORES = 16
_SC_WORKERS = _SC_CORES * _SC_SUBCORES
_B_PER_W = _BATCH // _SC_WORKERS


def _sc_gather_pairs(tbl2, xq):
    mesh = plsc.VectorSubcoreMesh(core_axis_name="c", subcore_axis_name="s")

    @pl.kernel(
        out_type=jax.ShapeDtypeStruct((_BATCH, 2 * _EMBED), tbl2.dtype),
        mesh=mesh,
        scratch_types=[
            pltpu.VMEM((_B_PER_W,), jnp.int32),
            pltpu.VMEM((_B_PER_W, 2 * _EMBED), jnp.float32),
            pltpu.SemaphoreType.DMA,
        ],
    )
    def gather_kernel(tbl_hbm, i_hbm, o_hbm, idx_v, rows_v, sem):
        wid = jax.lax.axis_index("s") * _SC_CORES + jax.lax.axis_index("c")
        base = wid * _B_PER_W
        pltpu.sync_copy(i_hbm.at[pl.ds(base, _B_PER_W)], idx_v)
        pltpu.async_copy(tbl_hbm.at[idx_v], rows_v, sem).wait()
        pltpu.sync_copy(rows_v, o_hbm.at[pl.ds(base, _B_PER_W)])

    return gather_kernel(tbl2, xq)


def _mm_body(par_ref, e2_ref, w_ref, b_ref, o_ref, e_ref):
    e2 = e2_ref[...]
    lo = jax.lax.slice(e2, (0, 0), (_BATCH, _EMBED))
    hi = jax.lax.slice(e2, (0, _EMBED), (_BATCH, 2 * _EMBED))
    e = jnp.where(par_ref[...] == 1, hi, lo)

    @pl.when(pl.program_id(0) == 0)
    def _():
        e_ref[...] = e

    o_ref[...] = jax.lax.dot_general(
        e, w_ref[...],
        (((1,), (1,)), ((), ())),
        preferred_element_type=jnp.float32,
    ) + b_ref[...]


def _tc_project(parity, e2, W, b):
    b2 = b.reshape(1, _VOCAB)
    grid = (pl.cdiv(_VOCAB, _V_TILE),)
    return pl.pallas_call(
        _mm_body,
        grid=grid,
        in_specs=[
            pl.BlockSpec((_BATCH, 1), lambda j: (0, 0)),
            pl.BlockSpec((_BATCH, 2 * _EMBED), lambda j: (0, 0)),
            pl.BlockSpec((_V_TILE, _EMBED), lambda j: (j, 0)),
            pl.BlockSpec((1, _V_TILE), lambda j: (0, j)),
        ],
        out_specs=[
            pl.BlockSpec((_BATCH, _V_TILE), lambda j: (0, j)),
            pl.BlockSpec((_BATCH, _EMBED), lambda j: (0, 0)),
        ],
        out_shape=[
            jax.ShapeDtypeStruct((_BATCH, _VOCAB), jnp.float32),
            jax.ShapeDtypeStruct((_BATCH, _EMBED), jnp.float32),
        ],
        compiler_params=pltpu.CompilerParams(
            dimension_semantics=("arbitrary",),
        ),
    )(parity, e2, W, b2)


def kernel(x, table, W, b):
    xi = x.astype(jnp.int32)
    tbl2 = table.reshape(_VOCAB // 2, 2 * _EMBED)
    e2 = _sc_gather_pairs(tbl2, xi >> 1)
    parity = (xi & 1).reshape(_BATCH, 1)
    logits, e = _tc_project(parity, e2, W, b)
    return (logits, e)


# speedup vs baseline: 1.0082x; 1.0082x over previous
import jax
import jax.numpy as jnp
from jax.experimental import pallas as pl
from jax.experimental.pallas import tpu as pltpu
from jax.experimental.pallas import tpu_sc as plsc

_VOCAB = 100000
_EMBED = 64
_BATCH = 1024

_V_TILE = 2048

_SC_CORES = 2
_SC_SUBCORES = 16
_SC_WORKERS = _SC_CORES * _SC_SUBCORES
_B_PER_W = _BATCH // _SC_WORKERS


def _sc_gather_pairs(tbl2, xq):
    mesh = plsc.VectorSubcoreMesh(core_axis_name="c", subcore_axis_name="s")

    @pl.kernel(
        out_type=jax.ShapeDtypeStruct((_BATCH, 2 * _EMBED), tbl2.dtype),
        mesh=mesh,
        scratch_types=[
            pltpu.VMEM((_B_PER_W,), jnp.int32),
            pltpu.VMEM((_B_PER_W, 2 * _EMBED), jnp.float32),
            pltpu.SemaphoreType.DMA,
        ],
    )
    def gather_kernel(tbl_hbm, i_hbm, o_hbm, idx_v, rows_v, sem):
        wid = jax.lax.axis_index("s") * _SC_CORES + jax.lax.axis_index("c")
        base = wid * _B_PER_W
        pltpu.sync_copy(i_hbm.at[pl.ds(base, _B_PER_W)], idx_v)
        pltpu.async_copy(tbl_hbm.at[idx_v], rows_v, sem).wait()
        pltpu.sync_copy(rows_v, o_hbm.at[pl.ds(base, _B_PER_W)])

    return gather_kernel(tbl2, xq)


def _mm_body(par_ref, e2_ref, w_ref, b_ref, o_ref, e_ref):
    e2 = e2_ref[...]
    lo = jax.lax.slice(e2, (0, 0), (_BATCH, _EMBED))
    hi = jax.lax.slice(e2, (0, _EMBED), (_BATCH, 2 * _EMBED))
    e = jnp.where(par_ref[...] == 1, hi, lo)

    @pl.when(pl.program_id(0) == 0)
    def _():
        e_ref[...] = e

    o_ref[...] = jax.lax.dot_general(
        e.astype(jnp.bfloat16), w_ref[...].astype(jnp.bfloat16),
        (((1,), (1,)), ((), ())),
        preferred_element_type=jnp.float32,
    ) + b_ref[...]


def _tc_project(parity, e2, W, b):
    b2 = b.reshape(1, _VOCAB)
    grid = (pl.cdiv(_VOCAB, _V_TILE),)
    return pl.pallas_call(
        _mm_body,
        grid=grid,
        in_specs=[
            pl.BlockSpec((_BATCH, 1), lambda j: (0, 0)),
            pl.BlockSpec((_BATCH, 2 * _EMBED), lambda j: (0, 0)),
            pl.BlockSpec((_V_TILE, _EMBED), lambda j: (j, 0)),
            pl.BlockSpec((1, _V_TILE), lambda j: (0, j)),
        ],
        out_specs=[
            pl.BlockSpec((_BATCH, _V_TILE), lambda j: (0, j)),
            pl.BlockSpec((_BATCH, _EMBED), lambda j: (0, 0)),
        ],
        out_shape=[
            jax.ShapeDtypeStruct((_BATCH, _VOCAB), jnp.float32),
            jax.ShapeDtypeStruct((_BATCH, _EMBED), jnp.float32),
        ],
        compiler_params=pltpu.CompilerParams(
            dimension_semantics=("parallel",),
        ),
    )(parity, e2, W, b2)


def kernel(x, table, W, b):
    xi = x.astype(jnp.int32)
    tbl2 = table.reshape(_VOCAB // 2, 2 * _EMBED)
    e2 = _sc_gather_pairs(tbl2, xi >> 1)
    parity = (xi & 1).reshape(_BATCH, 1)
    logits, e = _tc_project(parity, e2, W, b)
    return (logits, e)
